# baseline (speedup 1.0000x reference)
import jax
import jax.numpy as jnp
from jax.experimental import pallas as pl

B = 2048
L_MHC = 34
N_PEP = 24
MHC_DIM = 128
H = 64
PEP_DIM = 320
N_NODES = B * N_PEP
N_HE = N_NODES


def _lstm_layer(x, Wih, Whh, bih, bhh):
    Bb = x.shape[0]
    Hh = Whh.shape[1]
    def step(carry, xt):
        h, c = carry
        g = xt @ Wih.T + bih + h @ Whh.T + bhh
        i, f, gg, o = jnp.split(g, 4, axis=-1)
        i = jax.nn.sigmoid(i)
        f = jax.nn.sigmoid(f)
        gg = jnp.tanh(gg)
        o = jax.nn.sigmoid(o)
        c = f * c + i * gg
        h = o * jnp.tanh(c)
        return (h, c), h
    h0 = jnp.zeros((Bb, Hh), x.dtype)
    _, ys = jax.lax.scan(step, (h0, h0), jnp.swapaxes(x, 0, 1))
    return jnp.swapaxes(ys, 0, 1)


def _bilstm(x, params):
    out = x
    for l in (0, 1):
        fwd = _lstm_layer(out, params['l%d_Wih_f' % l], params['l%d_Whh_f' % l], params['l%d_bih_f' % l], params['l%d_bhh_f' % l])
        bwd = _lstm_layer(out[:, ::-1], params['l%d_Wih_b' % l], params['l%d_Whh_b' % l], params['l%d_bih_b' % l], params['l%d_bhh_b' % l])[:, ::-1]
        out = jnp.concatenate([fwd, bwd], axis=-1)
    return out


def _hconv(x, node_idx, he_idx, W, b):
    xt = x @ W.T
    ones_e = jnp.ones(node_idx.shape, xt.dtype)
    D = jax.ops.segment_sum(ones_e, node_idx, num_segments=N_NODES)
    Dinv = jnp.where(D > 0, 1.0 / D, 0.0)
    Be = jax.ops.segment_sum(ones_e, he_idx, num_segments=N_HE)
    Binv = jnp.where(Be > 0, 1.0 / Be, 0.0)
    m = jax.ops.segment_sum(xt[node_idx], he_idx, num_segments=N_HE) * Binv[:, None]
    out = jax.ops.segment_sum(m[he_idx], node_idx, num_segments=N_NODES) * Dinv[:, None]
    return out + b


def _copy_body(x_ref, o_ref):
    o_ref[...] = x_ref[...]


def kernel(x, edge_index, mhc_embed, batch, params):
    node_idx = edge_index[0]
    he_idx = edge_index[1]
    mhc_out = _bilstm(mhc_embed, params)
    h = _hconv(x, node_idx, he_idx, params['hc1_W'], params['hc1_b'])
    h = jax.nn.relu(h)
    h = _hconv(h, node_idx, he_idx, params['hc2_W'], params['hc2_b'])
    pep = h.reshape(mhc_out.shape[0], N_PEP, PEP_DIM)
    pep = pep @ params['pep_fc_W'].T + params['pep_fc_b']
    clus = jnp.einsum('bld,bpd->blp', mhc_out, pep)
    xc = clus.reshape(clus.shape[0], -1)
    xc = jax.nn.relu(xc @ params['fc1_W'].T + params['fc1_b'])
    out = xc @ params['out_W'].T + params['out_b']
    out = pl.pallas_call(
        _copy_body,
        out_shape=jax.ShapeDtypeStruct(out.shape, out.dtype),
    )(out)
    return out, clus


# speedup vs baseline: 1.1488x; 1.1488x over previous
import functools

import jax
import jax.numpy as jnp
from jax import lax
from jax.experimental import pallas as pl
from jax.experimental.pallas import tpu as pltpu
from jax.experimental.pallas import tpu_sc as plsc

B = 2048
L_MHC = 34
N_PEP = 24
MHC_DIM = 128
H = 64
PEP_DIM = 320
N_NODES = B * N_PEP
N_INC = 196608
NSLAB = 10
SLAB = 32
NC, NS = 2, 16
IDXW = 128
IDX_ROWS = N_INC // IDXW
ROWS_PER_TILE = IDX_ROWS // NS
STRIPE = N_NODES // NS

f32 = jnp.float32


def _sc_counts(nidx2d, hidx2d, ones128, zeros3072):
    mesh = plsc.VectorSubcoreMesh(
        core_axis_name="c", subcore_axis_name="s", num_cores=NC, num_subcores=NS)

    @functools.partial(
        pl.kernel,
        out_type=(jax.ShapeDtypeStruct((N_NODES,), f32),
                  jax.ShapeDtypeStruct((N_NODES,), f32)),
        mesh=mesh,
        scratch_types=[
            pltpu.VMEM((ROWS_PER_TILE, IDXW), jnp.int32),
            pltpu.VMEM((IDXW,), f32),
            pltpu.VMEM((STRIPE,), f32),
            pltpu.VMEM_SHARED((N_NODES,), f32),
        ],
    )
    def counts_kernel(nidx_hbm, hidx_hbm, ones_hbm, zeros_hbm,
                      dinv_hbm, binv_hbm, idx_v, ones_v, buf_v, acc_sh):
        c = lax.axis_index("c")
        s = lax.axis_index("s")

        pltpu.sync_copy(ones_hbm, ones_v)

        def one_dir(idx_hbm, out_hbm):
            pltpu.sync_copy(zeros_hbm, buf_v)
            pltpu.sync_copy(buf_v, acc_sh.at[pl.ds(s * STRIPE, STRIPE)])
            pltpu.sync_copy(idx_hbm.at[pl.ds(s * ROWS_PER_TILE, ROWS_PER_TILE)],
                            idx_v)
            plsc.subcore_barrier()

            def body(j, carry):
                pltpu.sync_copy(ones_v, acc_sh.at[idx_v.at[j]], add=True)
                return carry
            lax.fori_loop(0, ROWS_PER_TILE, body, 0)
            plsc.subcore_barrier()

            pltpu.sync_copy(acc_sh.at[pl.ds(s * STRIPE, STRIPE)], buf_v)

            def recip(i, carry):
                v = buf_v[pl.ds(i * 16, 16)]
                buf_v[pl.ds(i * 16, 16)] = jnp.where(v > 0.0, 1.0 / v, 0.0)
                return carry
            lax.fori_loop(0, STRIPE // 16, recip, 0)
            pltpu.sync_copy(buf_v, out_hbm.at[pl.ds(s * STRIPE, STRIPE)])

        @pl.when(c == 0)
        def _():
            one_dir(nidx_hbm, dinv_hbm)

        @pl.when(c == 1)
        def _():
            one_dir(hidx_hbm, binv_hbm)

    return counts_kernel(nidx2d, hidx2d, ones128, zeros3072)


def _sc_segsum(src_slab, gidx2d, sidx2d, zeros_stripe):
    mesh = plsc.VectorSubcoreMesh(
        core_axis_name="c", subcore_axis_name="s", num_cores=NC, num_subcores=NS)

    @functools.partial(
        pl.kernel,
        out_type=jax.ShapeDtypeStruct((NSLAB, N_NODES, SLAB), f32),
        mesh=mesh,
        scratch_types=[
            pltpu.VMEM((ROWS_PER_TILE, IDXW), jnp.int32),
            pltpu.VMEM((ROWS_PER_TILE, IDXW), jnp.int32),
            pltpu.VMEM((IDXW, SLAB), f32),
            pltpu.VMEM((IDXW, SLAB), f32),
            pltpu.VMEM_SHARED((N_NODES, SLAB), f32),
            pltpu.SemaphoreType.DMA,
            pltpu.SemaphoreType.DMA,
        ],
        compiler_params=pltpu.CompilerParams(use_tc_tiling_on_sc=False),
    )
    def segsum_kernel(src_hbm, gidx_hbm, sidx_hbm, zst_hbm, out_hbm,
                      gidx_v, sidx_v, rows0_v, rows1_v, acc_sh, sem0, sem1):
        c = lax.axis_index("c")
        s = lax.axis_index("s")

        pltpu.sync_copy(gidx_hbm.at[pl.ds(s * ROWS_PER_TILE, ROWS_PER_TILE)],
                        gidx_v)
        pltpu.sync_copy(sidx_hbm.at[pl.ds(s * ROWS_PER_TILE, ROWS_PER_TILE)],
                        sidx_v)

        def run_slab(slab, src_sl, out_sl):
            pltpu.sync_copy(zst_hbm,
                            acc_sh.at[pl.ds(s * STRIPE, STRIPE)])
            plsc.subcore_barrier()

            rows = (rows0_v, rows1_v)
            sems = (sem0, sem1)

            def gather_start(j, b):
                pltpu.make_async_copy(
                    src_sl.at[gidx_v.at[j]], rows[b], sems[b]).start()

            def gather_wait(j, b):
                pltpu.make_async_copy(
                    src_sl.at[gidx_v.at[j]], rows[b], sems[b]).wait()

            gather_start(0, 0)
            gather_start(1, 1)

            def body(it, carry):
                for bpar in (0, 1):
                    j = it * 2 + bpar
                    gather_wait(j, bpar)

                    @pl.when(j + 2 < ROWS_PER_TILE)
                    def _():
                        gather_start(j + 2, bpar)
                    pltpu.sync_copy(rows[bpar], acc_sh.at[sidx_v.at[j]],
                                    add=True)
                return carry
            lax.fori_loop(0, ROWS_PER_TILE // 2, body, 0)
            plsc.subcore_barrier()
            pltpu.sync_copy(acc_sh.at[pl.ds(s * STRIPE, STRIPE)],
                            out_sl.at[pl.ds(s * STRIPE, STRIPE)])
            plsc.subcore_barrier()

        for half in range(NSLAB // NC):
            for cc in range(NC):
                slab = half * NC + cc

                @pl.when(c == cc)
                def _(slab=slab):
                    run_slab(slab, src_hbm.at[slab], out_hbm.at[slab])

    return segsum_kernel(src_slab, gidx2d, sidx2d, zeros_stripe)


def _tc_mm_slab(x, wt, bm=512):
    def body(x_ref, w_ref, o_ref):
        acc = jnp.dot(x_ref[...], w_ref[...], preferred_element_type=f32)
        for sl in range(NSLAB):
            o_ref[sl] = acc[:, SLAB * sl:SLAB * (sl + 1)]

    return pl.pallas_call(
        body,
        grid=(N_NODES // bm,),
        in_specs=[pl.BlockSpec((bm, PEP_DIM), lambda i: (i, 0)),
                  pl.BlockSpec((PEP_DIM, PEP_DIM), lambda i: (0, 0))],
        out_specs=pl.BlockSpec((NSLAB, bm, SLAB), lambda i: (0, i, 0)),
        out_shape=jax.ShapeDtypeStruct((NSLAB, N_NODES, SLAB), f32),
    )(x, wt)


def _tc_scale(m, scale, br=4096):
    def body(m_ref, s_ref, o_ref):
        o_ref[...] = m_ref[...] * s_ref[...][None, :, None]

    return pl.pallas_call(
        body,
        grid=(NSLAB, N_NODES // br),
        in_specs=[pl.BlockSpec((1, br, SLAB), lambda sl, i: (sl, i, 0)),
                  pl.BlockSpec((br,), lambda sl, i: (i,))],
        out_specs=pl.BlockSpec((1, br, SLAB), lambda sl, i: (sl, i, 0)),
        out_shape=jax.ShapeDtypeStruct((NSLAB, N_NODES, SLAB), f32),
    )(m, scale)


def _tc_actmm_slab(g, dinv, b_slab, wt, bm=512):
    def body(g_ref, d_ref, b_ref, w_ref, o_ref):
        d = d_ref[...][:, None]
        parts = [jax.nn.relu(g_ref[sl] * d + b_ref[sl][None, :])
                 for sl in range(NSLAB)]
        h = jnp.concatenate(parts, axis=1)
        acc = jnp.dot(h, w_ref[...], preferred_element_type=f32)
        for sl in range(NSLAB):
            o_ref[sl] = acc[:, SLAB * sl:SLAB * (sl + 1)]

    return pl.pallas_call(
        body,
        grid=(N_NODES // bm,),
        in_specs=[pl.BlockSpec((NSLAB, bm, SLAB), lambda i: (0, i, 0)),
                  pl.BlockSpec((bm,), lambda i: (i,)),
                  pl.BlockSpec((NSLAB, SLAB), lambda i: (0, 0)),
                  pl.BlockSpec((PEP_DIM, PEP_DIM), lambda i: (0, 0))],
        out_specs=pl.BlockSpec((NSLAB, bm, SLAB), lambda i: (0, i, 0)),
        out_shape=jax.ShapeDtypeStruct((NSLAB, N_NODES, SLAB), f32),
    )(g, dinv, b_slab, wt)


def _tc_pepmm(g, dinv, b_slab, pep_wt, pep_b, bm=512):
    def body(g_ref, d_ref, b_ref, w_ref, pb_ref, o_ref):
        d = d_ref[...][:, None]
        parts = [g_ref[sl] * d + b_ref[sl][None, :] for sl in range(NSLAB)]
        h = jnp.concatenate(parts, axis=1)
        o_ref[...] = (jnp.dot(h, w_ref[...], preferred_element_type=f32)
                      + pb_ref[...][None, :])

    return pl.pallas_call(
        body,
        grid=(N_NODES // bm,),
        in_specs=[pl.BlockSpec((NSLAB, bm, SLAB), lambda i: (0, i, 0)),
                  pl.BlockSpec((bm,), lambda i: (i,)),
                  pl.BlockSpec((NSLAB, SLAB), lambda i: (0, 0)),
                  pl.BlockSpec((PEP_DIM, MHC_DIM), lambda i: (0, 0)),
                  pl.BlockSpec((MHC_DIM,), lambda i: (0,))],
        out_specs=pl.BlockSpec((bm, MHC_DIM), lambda i: (i, 0)),
        out_shape=jax.ShapeDtypeStruct((N_NODES, MHC_DIM), f32),
    )(g, dinv, b_slab, pep_wt, pep_b)


def _sigmoid(x):
    return 1.0 / (1.0 + jnp.exp(-x))


def _tc_bilstm(mhc, weights, bg=256):
    w_args = []
    for wih_t, whh_t, bsum in weights:
        w_args += [wih_t, whh_t, bsum]

    def body(x_ref, w00, w01, b0, w10, w11, b1, w20, w21, b2, w30, w31, b3,
             o_ref, gx_ref, ysf_ref, ysb_ref, l1_ref):
        ws = [(w00, w01, b0), (w10, w11, b1), (w20, w21, b2), (w30, w31, b3)]

        def run_dir(in2d, wih_t, whh_t, bsum, ys_ref, reverse):
            gx = jnp.dot(in2d, wih_t[...], preferred_element_type=f32)
            gx_ref[...] = (gx + bsum[...][None, :]).reshape(bg, L_MHC, 4 * H)

            def step(k, carry):
                h, cc = carry
                t = (L_MHC - 1 - k) if reverse else k
                g = gx_ref[:, pl.ds(t, 1), :].reshape(bg, 4 * H)
                g = g + jnp.dot(h, whh_t[...], preferred_element_type=f32)
                i = _sigmoid(g[:, 0:H])
                fgt = _sigmoid(g[:, H:2 * H])
                gg = jnp.tanh(g[:, 2 * H:3 * H])
                o = _sigmoid(g[:, 3 * H:4 * H])
                cc = fgt * cc + i * gg
                h = o * jnp.tanh(cc)
                ys_ref[:, pl.ds(t, 1), :] = h.reshape(bg, 1, H)
                return h, cc

            z = jnp.zeros((bg, H), f32)
            lax.fori_loop(0, L_MHC, step, (z, z))

        x2d = x_ref[...].reshape(bg * L_MHC, MHC_DIM)
        run_dir(x2d, *ws[0], ysf_ref, False)
        run_dir(x2d, *ws[1], ysb_ref, True)
        l1_ref[...] = jnp.concatenate([ysf_ref[...], ysb_ref[...]], axis=-1)
        l1_2d = l1_ref[...].reshape(bg * L_MHC, 2 * H)
        run_dir(l1_2d, *ws[2], ysf_ref, False)
        run_dir(l1_2d, *ws[3], ysb_ref, True)
        o_ref[...] = jnp.concatenate([ysf_ref[...], ysb_ref[...]], axis=-1)

    in_specs = [pl.BlockSpec((bg, L_MHC, MHC_DIM), lambda i: (i, 0, 0))]
    for wih_t, whh_t, bsum in weights:
        in_specs += [pl.BlockSpec(wih_t.shape, lambda i: tuple(0 for _ in wih_t.shape)),
                     pl.BlockSpec(whh_t.shape, lambda i: (0, 0)),
                     pl.BlockSpec(bsum.shape, lambda i: (0,))]

    return pl.pallas_call(
        body,
        grid=(B // bg,),
        in_specs=in_specs,
        out_specs=pl.BlockSpec((bg, L_MHC, MHC_DIM), lambda i: (i, 0, 0)),
        out_shape=jax.ShapeDtypeStruct((B, L_MHC, MHC_DIM), f32),
        scratch_shapes=[
            pltpu.VMEM((bg, L_MHC, 4 * H), f32),
            pltpu.VMEM((bg, L_MHC, H), f32),
            pltpu.VMEM((bg, L_MHC, H), f32),
            pltpu.VMEM((bg, L_MHC, 2 * H), f32),
        ],
    )(mhc, *w_args)


def _tc_head(mhc_out, pep, fc1_t, fc1_b, out_wt, out_b, g_blk=8):
    def body(m_ref, p_ref, f1_ref, fb_ref, ow_ref, ob_ref, clus_ref, out_ref):
        a = m_ref[...].reshape(g_blk * L_MHC, MHC_DIM)
        bmat = p_ref[...].reshape(g_blk * N_PEP, MHC_DIM)
        full = lax.dot_general(a, bmat, (((1,), (1,)), ((), ())),
                               preferred_element_type=f32)
        parts = [full[L_MHC * g:L_MHC * (g + 1), N_PEP * g:N_PEP * (g + 1)]
                 for g in range(g_blk)]
        clus = jnp.stack(parts, axis=0)
        clus_ref[...] = clus
        xcin = clus.reshape(g_blk, L_MHC * N_PEP)
        xc = jax.nn.relu(jnp.dot(xcin, f1_ref[...], preferred_element_type=f32)
                         + fb_ref[...][None, :])
        out_ref[...] = (jnp.dot(xc, ow_ref[...], preferred_element_type=f32)
                        + ob_ref[...][None, :])

    return pl.pallas_call(
        body,
        grid=(B // g_blk,),
        in_specs=[pl.BlockSpec((g_blk, L_MHC, MHC_DIM), lambda i: (i, 0, 0)),
                  pl.BlockSpec((g_blk, N_PEP, MHC_DIM), lambda i: (i, 0, 0)),
                  pl.BlockSpec((L_MHC * N_PEP, 16), lambda i: (0, 0)),
                  pl.BlockSpec((16,), lambda i: (0,)),
                  pl.BlockSpec((16, 1), lambda i: (0, 0)),
                  pl.BlockSpec((1,), lambda i: (0,))],
        out_specs=[pl.BlockSpec((g_blk, L_MHC, N_PEP), lambda i: (i, 0, 0)),
                   pl.BlockSpec((g_blk, 1), lambda i: (i, 0))],
        out_shape=[jax.ShapeDtypeStruct((B, L_MHC, N_PEP), f32),
                   jax.ShapeDtypeStruct((B, 1), f32)],
    )(mhc_out, pep, fc1_t, fc1_b, out_wt, out_b)


def kernel(x, edge_index, mhc_embed, batch, params):
    node_idx = edge_index[0]
    he_idx = edge_index[1]
    nidx2d = node_idx.reshape(IDX_ROWS, IDXW)
    hidx2d = he_idx.reshape(IDX_ROWS, IDXW)

    ones128 = jnp.ones((IDXW,), f32)
    zeros3072 = jnp.zeros((STRIPE,), f32)
    zeros_stripe = jnp.zeros((STRIPE, SLAB), f32)

    dinv, binv = _sc_counts(nidx2d, hidx2d, ones128, zeros3072)

    b1_slab = params['hc1_b'].reshape(NSLAB, SLAB)
    b2_slab = params['hc2_b'].reshape(NSLAB, SLAB)

    xt1 = _tc_mm_slab(x, params['hc1_W'].T)
    m1 = _sc_segsum(xt1, nidx2d, hidx2d, zeros_stripe)
    m1s = _tc_scale(m1, binv)
    g1 = _sc_segsum(m1s, hidx2d, nidx2d, zeros_stripe)
    xt2 = _tc_actmm_slab(g1, dinv, b1_slab, params['hc2_W'].T)
    m2 = _sc_segsum(xt2, nidx2d, hidx2d, zeros_stripe)
    m2s = _tc_scale(m2, binv)
    g2 = _sc_segsum(m2s, hidx2d, nidx2d, zeros_stripe)
    pep_flat = _tc_pepmm(g2, dinv, b2_slab, params['pep_fc_W'].T,
                         params['pep_fc_b'])
    pep = pep_flat.reshape(B, N_PEP, MHC_DIM)

    weights = []
    for l in (0, 1):
        for d in ('f', 'b'):
            wih = params['l%d_Wih_%s' % (l, d)]
            whh = params['l%d_Whh_%s' % (l, d)]
            bsum = params['l%d_bih_%s' % (l, d)] + params['l%d_bhh_%s' % (l, d)]
            weights.append((wih.T, whh.T, bsum))
    mhc_out = _tc_bilstm(mhc_embed, weights)

    clus, out = _tc_head(mhc_out, pep, params['fc1_W'].T, params['fc1_b'],
                         params['out_W'].T, params['out_b'])
    return out, clus


# speedup vs baseline: 1.4131x; 1.2301x over previous
import functools

import jax
import jax.numpy as jnp
from jax import lax
from jax.experimental import pallas as pl
from jax.experimental.pallas import tpu as pltpu
from jax.experimental.pallas import tpu_sc as plsc

B = 2048
L_MHC = 34
N_PEP = 24
MHC_DIM = 128
H = 64
PEP_DIM = 320
N_NODES = B * N_PEP
N4 = N_NODES // 4
N_INC = 196608
NSLAB = 10
SLAB = 32
NC, NS = 2, 16
IDXW = 128
IDX_ROWS = N_INC // IDXW
ROWS_PER_TILE = IDX_ROWS // NS
STRIPE = N_NODES // NS

f32 = jnp.float32
i32 = jnp.int32


def _sc_counts(nidx2d, hidx2d, ones128, zeros3072):
    mesh = plsc.VectorSubcoreMesh(
        core_axis_name="c", subcore_axis_name="s", num_cores=NC, num_subcores=NS)

    @functools.partial(
        pl.kernel,
        out_type=(jax.ShapeDtypeStruct((N_NODES,), f32),
                  jax.ShapeDtypeStruct((N_NODES,), f32)),
        mesh=mesh,
        scratch_types=[
            pltpu.VMEM((ROWS_PER_TILE, IDXW), i32),
            pltpu.VMEM((IDXW,), f32),
            pltpu.VMEM((STRIPE,), f32),
            pltpu.VMEM_SHARED((N_NODES,), f32),
        ],
        compiler_params=pltpu.CompilerParams(use_tc_tiling_on_sc=False),
    )
    def counts_kernel(nidx_hbm, hidx_hbm, ones_hbm, zeros_hbm,
                      dinv_hbm, binv_hbm, idx_v, ones_v, buf_v, acc_sh):
        c = lax.axis_index("c")
        s = lax.axis_index("s")

        pltpu.sync_copy(ones_hbm, ones_v)

        def one_dir(idx_hbm, out_hbm):
            pltpu.sync_copy(zeros_hbm, buf_v)
            pltpu.sync_copy(buf_v, acc_sh.at[pl.ds(s * STRIPE, STRIPE)])
            pltpu.sync_copy(idx_hbm.at[pl.ds(s * ROWS_PER_TILE, ROWS_PER_TILE)],
                            idx_v)
            plsc.subcore_barrier()

            def body(j, carry):
                pltpu.sync_copy(ones_v, acc_sh.at[idx_v.at[j]], add=True)
                return carry
            lax.fori_loop(0, ROWS_PER_TILE, body, 0)
            plsc.subcore_barrier()

            pltpu.sync_copy(acc_sh.at[pl.ds(s * STRIPE, STRIPE)], buf_v)

            def recip(i, carry):
                v = buf_v[pl.ds(i * 16, 16)]
                buf_v[pl.ds(i * 16, 16)] = jnp.where(v > 0.0, 1.0 / v, 0.0)
                return carry
            lax.fori_loop(0, STRIPE // 16, recip, 0)
            pltpu.sync_copy(buf_v, out_hbm.at[pl.ds(s * STRIPE, STRIPE)])

        @pl.when(c == 0)
        def _():
            one_dir(nidx_hbm, dinv_hbm)

        @pl.when(c == 1)
        def _():
            one_dir(hidx_hbm, binv_hbm)

    return counts_kernel(nidx2d, hidx2d, ones128, zeros3072)


def _sc_segsum(src_slab, gidx2d, sidx2d, zeros_stripe):
    mesh = plsc.VectorSubcoreMesh(
        core_axis_name="c", subcore_axis_name="s", num_cores=NC, num_subcores=NS)

    @functools.partial(
        pl.kernel,
        out_type=jax.ShapeDtypeStruct((NSLAB, N_NODES, SLAB), f32),
        mesh=mesh,
        scratch_types=[
            pltpu.VMEM((ROWS_PER_TILE, IDXW), i32),
            pltpu.VMEM((ROWS_PER_TILE, IDXW), i32),
            pltpu.VMEM((IDXW, SLAB), f32),
            pltpu.VMEM((IDXW, SLAB), f32),
            pltpu.VMEM_SHARED((N_NODES, SLAB), f32),
            pltpu.SemaphoreType.DMA,
            pltpu.SemaphoreType.DMA,
        ],
        compiler_params=pltpu.CompilerParams(use_tc_tiling_on_sc=False),
    )
    def segsum_kernel(src_hbm, gidx_hbm, sidx_hbm, zst_hbm, out_hbm,
                      gidx_v, sidx_v, rows0_v, rows1_v, acc_sh, sem0, sem1):
        c = lax.axis_index("c")
        s = lax.axis_index("s")

        pltpu.sync_copy(gidx_hbm.at[pl.ds(s * ROWS_PER_TILE, ROWS_PER_TILE)],
                        gidx_v)
        pltpu.sync_copy(sidx_hbm.at[pl.ds(s * ROWS_PER_TILE, ROWS_PER_TILE)],
                        sidx_v)

        def run_slab(src_sl, out_sl):
            pltpu.sync_copy(zst_hbm, acc_sh.at[pl.ds(s * STRIPE, STRIPE)])
            plsc.subcore_barrier()

            rows = (rows0_v, rows1_v)
            sems = (sem0, sem1)

            def gather_start(j, b):
                pltpu.make_async_copy(
                    src_sl.at[gidx_v.at[j]], rows[b], sems[b]).start()

            def gather_wait(j, b):
                pltpu.make_async_copy(
                    src_sl.at[gidx_v.at[j]], rows[b], sems[b]).wait()

            gather_start(0, 0)
            gather_start(1, 1)

            def body(it, carry):
                for bpar in (0, 1):
                    j = it * 2 + bpar
                    gather_wait(j, bpar)

                    @pl.when(j + 2 < ROWS_PER_TILE)
                    def _():
                        gather_start(j + 2, bpar)
                    pltpu.sync_copy(rows[bpar], acc_sh.at[sidx_v.at[j]],
                                    add=True)
                return carry
            lax.fori_loop(0, ROWS_PER_TILE // 2, body, 0)
            plsc.subcore_barrier()
            pltpu.sync_copy(acc_sh.at[pl.ds(s * STRIPE, STRIPE)],
                            out_sl.at[pl.ds(s * STRIPE, STRIPE)])
            plsc.subcore_barrier()

        for half in range(NSLAB // NC):
            for cc in range(NC):
                slab = half * NC + cc

                @pl.when(c == cc)
                def _(slab=slab):
                    run_slab(src_hbm.at[slab], out_hbm.at[slab])

    return segsum_kernel(src_slab, gidx2d, sidx2d, zeros_stripe)


def _tc_mm_slab(x, wt, bm=512):
    def body(x_ref, w_ref, o_ref):
        acc = jnp.dot(x_ref[...], w_ref[...], preferred_element_type=f32)
        for sl in range(NSLAB):
            o_ref[sl] = acc[:, SLAB * sl:SLAB * (sl + 1)]

    return pl.pallas_call(
        body,
        grid=(N_NODES // bm,),
        in_specs=[pl.BlockSpec((bm, PEP_DIM), lambda i: (i, 0)),
                  pl.BlockSpec((PEP_DIM, PEP_DIM), lambda i: (0, 0))],
        out_specs=pl.BlockSpec((NSLAB, bm, SLAB), lambda i: (0, i, 0)),
        out_shape=jax.ShapeDtypeStruct((NSLAB, N_NODES, SLAB), f32),
    )(x, wt)


def _tc_scale(m, scale_f, br4=1024):
    def body(m_ref, s_ref, o_ref):
        sf = s_ref[...]
        srow = jnp.concatenate(
            [jnp.broadcast_to(sf[:, q:q + 1], (br4, SLAB)) for q in range(4)],
            axis=1)
        for sl in range(NSLAB):
            o_ref[sl] = m_ref[sl] * srow

    return pl.pallas_call(
        body,
        grid=(N4 // br4,),
        in_specs=[pl.BlockSpec((NSLAB, br4, 4 * SLAB), lambda i: (0, i, 0)),
                  pl.BlockSpec((br4, 4), lambda i: (i, 0))],
        out_specs=pl.BlockSpec((NSLAB, br4, 4 * SLAB), lambda i: (0, i, 0)),
        out_shape=jax.ShapeDtypeStruct((NSLAB, N4, 4 * SLAB), f32),
    )(m, scale_f)


def _tc_actmm_slab(g, dinv, b_slab, wt, bm=512):
    def body(g_ref, d_ref, b_ref, w_ref, o_ref):
        d = d_ref[...][:, None]
        parts = [jax.nn.relu(g_ref[sl] * d + b_ref[sl][None, :])
                 for sl in range(NSLAB)]
        h = jnp.concatenate(parts, axis=1)
        acc = jnp.dot(h, w_ref[...], preferred_element_type=f32)
        for sl in range(NSLAB):
            o_ref[sl] = acc[:, SLAB * sl:SLAB * (sl + 1)]

    return pl.pallas_call(
        body,
        grid=(N_NODES // bm,),
        in_specs=[pl.BlockSpec((NSLAB, bm, SLAB), lambda i: (0, i, 0)),
                  pl.BlockSpec((bm,), lambda i: (i,)),
                  pl.BlockSpec((NSLAB, SLAB), lambda i: (0, 0)),
                  pl.BlockSpec((PEP_DIM, PEP_DIM), lambda i: (0, 0))],
        out_specs=pl.BlockSpec((NSLAB, bm, SLAB), lambda i: (0, i, 0)),
        out_shape=jax.ShapeDtypeStruct((NSLAB, N_NODES, SLAB), f32),
    )(g, dinv, b_slab, wt)


def _tc_pepmm(g, dinv, b_slab, pep_wt, pep_b, bm=512):
    def body(g_ref, d_ref, b_ref, w_ref, pb_ref, o_ref):
        d = d_ref[...][:, None]
        parts = [g_ref[sl] * d + b_ref[sl][None, :] for sl in range(NSLAB)]
        h = jnp.concatenate(parts, axis=1)
        o_ref[...] = (jnp.dot(h, w_ref[...], preferred_element_type=f32)
                      + pb_ref[...][None, :])

    return pl.pallas_call(
        body,
        grid=(N_NODES // bm,),
        in_specs=[pl.BlockSpec((NSLAB, bm, SLAB), lambda i: (0, i, 0)),
                  pl.BlockSpec((bm,), lambda i: (i,)),
                  pl.BlockSpec((NSLAB, SLAB), lambda i: (0, 0)),
                  pl.BlockSpec((PEP_DIM, MHC_DIM), lambda i: (0, 0)),
                  pl.BlockSpec((MHC_DIM,), lambda i: (0,))],
        out_specs=pl.BlockSpec((bm, MHC_DIM), lambda i: (i, 0)),
        out_shape=jax.ShapeDtypeStruct((N_NODES, MHC_DIM), f32),
    )(g, dinv, b_slab, pep_wt, pep_b)


def _sigmoid(x):
    return 1.0 / (1.0 + jnp.exp(-x))


def _tc_bilstm(mhc, weights, bg=256):
    w_args = []
    for wih_t, whh_t, bsum in weights:
        w_args += [wih_t, whh_t, bsum]

    def body(x_ref, w00, w01, b0, w10, w11, b1, w20, w21, b2, w30, w31, b3,
             o_ref, gx_ref, ysf_ref, ysb_ref, l1_ref):
        ws = [(w00, w01, b0), (w10, w11, b1), (w20, w21, b2), (w30, w31, b3)]

        def run_dir(in2d, wih_t, whh_t, bsum, ys_ref, reverse):
            gx = jnp.dot(in2d, wih_t[...], preferred_element_type=f32)
            gx_ref[...] = (gx + bsum[...][None, :]).reshape(bg, L_MHC, 4 * H)

            def step(k, carry):
                h, cc = carry
                t = (L_MHC - 1 - k) if reverse else k
                g = gx_ref[:, pl.ds(t, 1), :].reshape(bg, 4 * H)
                g = g + jnp.dot(h, whh_t[...], preferred_element_type=f32)
                i = _sigmoid(g[:, 0:H])
                fgt = _sigmoid(g[:, H:2 * H])
                gg = jnp.tanh(g[:, 2 * H:3 * H])
                o = _sigmoid(g[:, 3 * H:4 * H])
                cc = fgt * cc + i * gg
                h = o * jnp.tanh(cc)
                ys_ref[:, pl.ds(t, 1), :] = h.reshape(bg, 1, H)
                return h, cc

            z = jnp.zeros((bg, H), f32)
            lax.fori_loop(0, L_MHC, step, (z, z))

        x2d = x_ref[...].reshape(bg * L_MHC, MHC_DIM)
        run_dir(x2d, *ws[0], ysf_ref, False)
        run_dir(x2d, *ws[1], ysb_ref, True)
        l1_ref[...] = jnp.concatenate([ysf_ref[...], ysb_ref[...]], axis=-1)
        l1_2d = l1_ref[...].reshape(bg * L_MHC, 2 * H)
        run_dir(l1_2d, *ws[2], ysf_ref, False)
        run_dir(l1_2d, *ws[3], ysb_ref, True)
        o_ref[...] = jnp.concatenate([ysf_ref[...], ysb_ref[...]], axis=-1)

    in_specs = [pl.BlockSpec((bg, L_MHC, MHC_DIM), lambda i: (i, 0, 0))]
    for wih_t, whh_t, bsum in weights:
        in_specs += [pl.BlockSpec(wih_t.shape, lambda i: (0, 0)),
                     pl.BlockSpec(whh_t.shape, lambda i: (0, 0)),
                     pl.BlockSpec(bsum.shape, lambda i: (0,))]

    return pl.pallas_call(
        body,
        grid=(B // bg,),
        in_specs=in_specs,
        out_specs=pl.BlockSpec((bg, L_MHC, MHC_DIM), lambda i: (i, 0, 0)),
        out_shape=jax.ShapeDtypeStruct((B, L_MHC, MHC_DIM), f32),
        scratch_shapes=[
            pltpu.VMEM((bg, L_MHC, 4 * H), f32),
            pltpu.VMEM((bg, L_MHC, H), f32),
            pltpu.VMEM((bg, L_MHC, H), f32),
            pltpu.VMEM((bg, L_MHC, 2 * H), f32),
        ],
    )(mhc, *w_args)


def _tc_head(mhc_out, pep, fc1_t, fc1_b, out_wt, out_b, g_blk=8):
    def body(m_ref, p_ref, f1_ref, fb_ref, ow_ref, ob_ref, clus_ref, out_ref):
        a = m_ref[...].reshape(g_blk * L_MHC, MHC_DIM)
        bmat = p_ref[...].reshape(g_blk * N_PEP, MHC_DIM)
        full = lax.dot_general(a, bmat, (((1,), (1,)), ((), ())),
                               preferred_element_type=f32)
        parts = [full[L_MHC * g:L_MHC * (g + 1), N_PEP * g:N_PEP * (g + 1)]
                 for g in range(g_blk)]
        clus = jnp.stack(parts, axis=0)
        clus_ref[...] = clus
        xcin = clus.reshape(g_blk, L_MHC * N_PEP)
        xc = jax.nn.relu(jnp.dot(xcin, f1_ref[...], preferred_element_type=f32)
                         + fb_ref[...][None, :])
        out_ref[...] = (jnp.dot(xc, ow_ref[...], preferred_element_type=f32)
                        + ob_ref[...][None, :])

    return pl.pallas_call(
        body,
        grid=(B // g_blk,),
        in_specs=[pl.BlockSpec((g_blk, L_MHC, MHC_DIM), lambda i: (i, 0, 0)),
                  pl.BlockSpec((g_blk, N_PEP, MHC_DIM), lambda i: (i, 0, 0)),
                  pl.BlockSpec((L_MHC * N_PEP, 16), lambda i: (0, 0)),
                  pl.BlockSpec((16,), lambda i: (0,)),
                  pl.BlockSpec((16, 1), lambda i: (0, 0)),
                  pl.BlockSpec((1,), lambda i: (0,))],
        out_specs=[pl.BlockSpec((g_blk, L_MHC, N_PEP), lambda i: (i, 0, 0)),
                   pl.BlockSpec((g_blk, 1), lambda i: (i, 0))],
        out_shape=[jax.ShapeDtypeStruct((B, L_MHC, N_PEP), f32),
                   jax.ShapeDtypeStruct((B, 1), f32)],
    )(mhc_out, pep, fc1_t, fc1_b, out_wt, out_b)


def _fold(a):
    return a.reshape(NSLAB, N4, 4 * SLAB)


def _unfold(a):
    return a.reshape(NSLAB, N_NODES, SLAB)


def kernel(x, edge_index, mhc_embed, batch, params):
    node_idx = edge_index[0]
    he_idx = edge_index[1]
    nidx2d = node_idx.reshape(IDX_ROWS, IDXW)
    hidx2d = he_idx.reshape(IDX_ROWS, IDXW)

    ones128 = jnp.ones((IDXW,), f32)
    zeros3072 = jnp.zeros((STRIPE,), f32)
    zeros_stripe = jnp.zeros((STRIPE, SLAB), f32)

    dinv, binv = _sc_counts(nidx2d, hidx2d, ones128, zeros3072)
    binv_f = binv.reshape(N4, 4)

    b1_slab = params['hc1_b'].reshape(NSLAB, SLAB)
    b2_slab = params['hc2_b'].reshape(NSLAB, SLAB)

    xt1 = _tc_mm_slab(x, params['hc1_W'].T)
    m1 = _sc_segsum(xt1, nidx2d, hidx2d, zeros_stripe)
    m1s = _tc_scale(_fold(m1), binv_f)
    g1 = _sc_segsum(_unfold(m1s), hidx2d, nidx2d, zeros_stripe)
    xt2 = _tc_actmm_slab(g1, dinv, b1_slab, params['hc2_W'].T)
    m2 = _sc_segsum(xt2, nidx2d, hidx2d, zeros_stripe)
    m2s = _tc_scale(_fold(m2), binv_f)
    g2 = _sc_segsum(_unfold(m2s), hidx2d, nidx2d, zeros_stripe)
    pep_flat = _tc_pepmm(g2, dinv, b2_slab, params['pep_fc_W'].T,
                         params['pep_fc_b'])
    pep = pep_flat.reshape(B, N_PEP, MHC_DIM)

    weights = []
    for l in (0, 1):
        for d in ('f', 'b'):
            wih = params['l%d_Wih_%s' % (l, d)]
            whh = params['l%d_Whh_%s' % (l, d)]
            bsum = params['l%d_bih_%s' % (l, d)] + params['l%d_bhh_%s' % (l, d)]
            weights.append((wih.T, whh.T, bsum))
    mhc_out = _tc_bilstm(mhc_embed, weights)

    clus, out = _tc_head(mhc_out, pep, params['fc1_W'].T, params['fc1_b'],
                         params['out_W'].T, params['out_b'])
    return out, clus


# speedup vs baseline: 1.8315x; 1.2961x over previous
import functools

import jax
import jax.numpy as jnp
from jax import lax
from jax.experimental import pallas as pl
from jax.experimental.pallas import tpu as pltpu
from jax.experimental.pallas import tpu_sc as plsc

B = 2048
L_MHC = 34
N_PEP = 24
MHC_DIM = 128
H = 64
PEP_DIM = 320
N_NODES = B * N_PEP
N4 = N_NODES // 4
N_INC = 196608
NSLAB = 10
SLAB = 32
NC, NS = 2, 16
IDXW = 128
IDX_ROWS = N_INC // IDXW
ROWS_PER_TILE = IDX_ROWS // NS
STRIPE = N_NODES // NS

f32 = jnp.float32
i32 = jnp.int32


def _sc_counts(nidx2d, hidx2d, ones128, zeros3072):
    mesh = plsc.VectorSubcoreMesh(
        core_axis_name="c", subcore_axis_name="s", num_cores=NC, num_subcores=NS)

    @functools.partial(
        pl.kernel,
        out_type=(jax.ShapeDtypeStruct((N_NODES,), f32),
                  jax.ShapeDtypeStruct((N_NODES,), f32)),
        mesh=mesh,
        scratch_types=[
            pltpu.VMEM((ROWS_PER_TILE, IDXW), i32),
            pltpu.VMEM((IDXW,), f32),
            pltpu.VMEM((STRIPE,), f32),
            pltpu.VMEM_SHARED((N_NODES,), f32),
        ],
        compiler_params=pltpu.CompilerParams(use_tc_tiling_on_sc=False),
    )
    def counts_kernel(nidx_hbm, hidx_hbm, ones_hbm, zeros_hbm,
                      dinv_hbm, binv_hbm, idx_v, ones_v, buf_v, acc_sh):
        c = lax.axis_index("c")
        s = lax.axis_index("s")

        pltpu.sync_copy(ones_hbm, ones_v)

        def one_dir(idx_hbm, out_hbm):
            pltpu.sync_copy(zeros_hbm, buf_v)
            pltpu.sync_copy(buf_v, acc_sh.at[pl.ds(s * STRIPE, STRIPE)])
            pltpu.sync_copy(idx_hbm.at[pl.ds(s * ROWS_PER_TILE, ROWS_PER_TILE)],
                            idx_v)
            plsc.subcore_barrier()

            def body(j, carry):
                pltpu.sync_copy(ones_v, acc_sh.at[idx_v.at[j]], add=True)
                return carry
            lax.fori_loop(0, ROWS_PER_TILE, body, 0)
            plsc.subcore_barrier()

            pltpu.sync_copy(acc_sh.at[pl.ds(s * STRIPE, STRIPE)], buf_v)

            def recip(i, carry):
                v = buf_v[pl.ds(i * 16, 16)]
                buf_v[pl.ds(i * 16, 16)] = jnp.where(v > 0.0, 1.0 / v, 0.0)
                return carry
            lax.fori_loop(0, STRIPE // 16, recip, 0)
            pltpu.sync_copy(buf_v, out_hbm.at[pl.ds(s * STRIPE, STRIPE)])

        @pl.when(c == 0)
        def _():
            one_dir(nidx_hbm, dinv_hbm)

        @pl.when(c == 1)
        def _():
            one_dir(hidx_hbm, binv_hbm)

    return counts_kernel(nidx2d, hidx2d, ones128, zeros3072)


def _sc_segsum(src_slab, gidx2d, sidx2d, zeros_stripe):
    mesh = plsc.VectorSubcoreMesh(
        core_axis_name="c", subcore_axis_name="s", num_cores=NC, num_subcores=NS)

    @functools.partial(
        pl.kernel,
        out_type=jax.ShapeDtypeStruct((NSLAB, N_NODES, SLAB), f32),
        mesh=mesh,
        scratch_types=[
            pltpu.VMEM((ROWS_PER_TILE, IDXW), i32),
            pltpu.VMEM((ROWS_PER_TILE, IDXW), i32),
            pltpu.VMEM((IDXW, SLAB), f32),
            pltpu.VMEM((IDXW, SLAB), f32),
            pltpu.VMEM_SHARED((N_NODES, SLAB), f32),
            pltpu.SemaphoreType.DMA,
            pltpu.SemaphoreType.DMA,
        ],
        compiler_params=pltpu.CompilerParams(use_tc_tiling_on_sc=False),
    )
    def segsum_kernel(src_hbm, gidx_hbm, sidx_hbm, zst_hbm, out_hbm,
                      gidx_v, sidx_v, rows0_v, rows1_v, acc_sh, sem0, sem1):
        c = lax.axis_index("c")
        s = lax.axis_index("s")

        pltpu.sync_copy(gidx_hbm.at[pl.ds(s * ROWS_PER_TILE, ROWS_PER_TILE)],
                        gidx_v)
        pltpu.sync_copy(sidx_hbm.at[pl.ds(s * ROWS_PER_TILE, ROWS_PER_TILE)],
                        sidx_v)

        def run_slab(src_sl, out_sl):
            pltpu.sync_copy(zst_hbm, acc_sh.at[pl.ds(s * STRIPE, STRIPE)])
            plsc.subcore_barrier()

            rows = (rows0_v, rows1_v)
            sems = (sem0, sem1)

            def gather_start(j, b):
                pltpu.make_async_copy(
                    src_sl.at[gidx_v.at[j]], rows[b], sems[b]).start()

            def gather_wait(j, b):
                pltpu.make_async_copy(
                    src_sl.at[gidx_v.at[j]], rows[b], sems[b]).wait()

            gather_start(0, 0)
            gather_start(1, 1)

            def body(it, carry):
                for bpar in (0, 1):
                    j = it * 2 + bpar
                    gather_wait(j, bpar)

                    @pl.when(j + 2 < ROWS_PER_TILE)
                    def _():
                        gather_start(j + 2, bpar)
                    pltpu.sync_copy(rows[bpar], acc_sh.at[sidx_v.at[j]],
                                    add=True)
                return carry
            lax.fori_loop(0, ROWS_PER_TILE // 2, body, 0)
            plsc.subcore_barrier()
            pltpu.sync_copy(acc_sh.at[pl.ds(s * STRIPE, STRIPE)],
                            out_sl.at[pl.ds(s * STRIPE, STRIPE)])
            plsc.subcore_barrier()

        for half in range(NSLAB // NC):
            for cc in range(NC):
                slab = half * NC + cc

                @pl.when(c == cc)
                def _(slab=slab):
                    run_slab(src_hbm.at[slab], out_hbm.at[slab])

    return segsum_kernel(src_slab, gidx2d, sidx2d, zeros_stripe)


def _tc_mm_slab(x, wt, bm=512):
    def body(x_ref, w_ref, o_ref):
        acc = jnp.dot(x_ref[...], w_ref[...], preferred_element_type=f32)
        for sl in range(NSLAB):
            o_ref[sl] = acc[:, SLAB * sl:SLAB * (sl + 1)]

    return pl.pallas_call(
        body,
        grid=(N_NODES // bm,),
        in_specs=[pl.BlockSpec((bm, PEP_DIM), lambda i: (i, 0)),
                  pl.BlockSpec((PEP_DIM, PEP_DIM), lambda i: (0, 0))],
        out_specs=pl.BlockSpec((NSLAB, bm, SLAB), lambda i: (0, i, 0)),
        out_shape=jax.ShapeDtypeStruct((NSLAB, N_NODES, SLAB), f32),
    )(x, wt)


def _tc_scale(m, scale_f, br4=1024):
    def body(m_ref, s_ref, o_ref):
        sf = s_ref[...]
        srow = jnp.concatenate(
            [jnp.broadcast_to(sf[:, q:q + 1], (br4, SLAB)) for q in range(4)],
            axis=1)
        for sl in range(NSLAB):
            o_ref[sl] = m_ref[sl] * srow

    return pl.pallas_call(
        body,
        grid=(N4 // br4,),
        in_specs=[pl.BlockSpec((NSLAB, br4, 4 * SLAB), lambda i: (0, i, 0)),
                  pl.BlockSpec((br4, 4), lambda i: (i, 0))],
        out_specs=pl.BlockSpec((NSLAB, br4, 4 * SLAB), lambda i: (0, i, 0)),
        out_shape=jax.ShapeDtypeStruct((NSLAB, N4, 4 * SLAB), f32),
    )(m, scale_f)


def _tc_actmm_slab(g, dinv, b_slab, wt, bm=512):
    def body(g_ref, d_ref, b_ref, w_ref, o_ref):
        d = d_ref[...][:, None]
        parts = [jax.nn.relu(g_ref[sl] * d + b_ref[sl][None, :])
                 for sl in range(NSLAB)]
        h = jnp.concatenate(parts, axis=1)
        acc = jnp.dot(h, w_ref[...], preferred_element_type=f32)
        for sl in range(NSLAB):
            o_ref[sl] = acc[:, SLAB * sl:SLAB * (sl + 1)]

    return pl.pallas_call(
        body,
        grid=(N_NODES // bm,),
        in_specs=[pl.BlockSpec((NSLAB, bm, SLAB), lambda i: (0, i, 0)),
                  pl.BlockSpec((bm,), lambda i: (i,)),
                  pl.BlockSpec((NSLAB, SLAB), lambda i: (0, 0)),
                  pl.BlockSpec((PEP_DIM, PEP_DIM), lambda i: (0, 0))],
        out_specs=pl.BlockSpec((NSLAB, bm, SLAB), lambda i: (0, i, 0)),
        out_shape=jax.ShapeDtypeStruct((NSLAB, N_NODES, SLAB), f32),
    )(g, dinv, b_slab, wt)


def _tc_pepmm(g, dinv, b_slab, pep_wt, pep_b, bm=512):
    def body(g_ref, d_ref, b_ref, w_ref, pb_ref, o_ref):
        d = d_ref[...][:, None]
        parts = [g_ref[sl] * d + b_ref[sl][None, :] for sl in range(NSLAB)]
        h = jnp.concatenate(parts, axis=1)
        o_ref[...] = (jnp.dot(h, w_ref[...], preferred_element_type=f32)
                      + pb_ref[...][None, :])

    return pl.pallas_call(
        body,
        grid=(N_NODES // bm,),
        in_specs=[pl.BlockSpec((NSLAB, bm, SLAB), lambda i: (0, i, 0)),
                  pl.BlockSpec((bm,), lambda i: (i,)),
                  pl.BlockSpec((NSLAB, SLAB), lambda i: (0, 0)),
                  pl.BlockSpec((PEP_DIM, MHC_DIM), lambda i: (0, 0)),
                  pl.BlockSpec((MHC_DIM,), lambda i: (0,))],
        out_specs=pl.BlockSpec((bm, MHC_DIM), lambda i: (i, 0)),
        out_shape=jax.ShapeDtypeStruct((N_NODES, MHC_DIM), f32),
    )(g, dinv, b_slab, pep_wt, pep_b)


def _sigmoid(x):
    return 1.0 / (1.0 + jnp.exp(-x))


def _tc_bilstm(mhc, weights, bg=256):
    w_args = []
    for tup in weights:
        w_args += list(tup)

    def body(x_ref, wf0, wb0, wh0, bf0, bb0, wf1, wb1, wh1, bf1, bb1,
             o_ref, gxf_ref, gxb_ref, ysf_ref, ysb_ref, l1_ref):
        ws = [(wf0, wb0, wh0, bf0, bb0), (wf1, wb1, wh1, bf1, bb1)]

        def run_layer(in2d, wihf_t, wihb_t, whh_fb, bsf, bsb):
            gxf = jnp.dot(in2d, wihf_t[...], preferred_element_type=f32)
            gxf_ref[...] = (gxf + bsf[...][None, :]).reshape(bg, L_MHC, 4 * H)
            gxb = jnp.dot(in2d, wihb_t[...], preferred_element_type=f32)
            gxb_ref[...] = (gxb + bsb[...][None, :]).reshape(bg, L_MHC, 4 * H)

            def step(k, carry):
                h2, cf, cb = carry
                tf = k
                tb = L_MHC - 1 - k
                gf = gxf_ref[:, pl.ds(tf, 1), :].reshape(bg, 4 * H)
                gb = gxb_ref[:, pl.ds(tb, 1), :].reshape(bg, 4 * H)
                g = jnp.concatenate([gf, gb], axis=1)
                g = g + jnp.dot(h2, whh_fb[...], preferred_element_type=f32)
                i_f = _sigmoid(g[:, 0:H])
                f_f = _sigmoid(g[:, H:2 * H])
                g_f = jnp.tanh(g[:, 2 * H:3 * H])
                o_f = _sigmoid(g[:, 3 * H:4 * H])
                i_b = _sigmoid(g[:, 4 * H:5 * H])
                f_b = _sigmoid(g[:, 5 * H:6 * H])
                g_b = jnp.tanh(g[:, 6 * H:7 * H])
                o_b = _sigmoid(g[:, 7 * H:8 * H])
                cf = f_f * cf + i_f * g_f
                hf = o_f * jnp.tanh(cf)
                cb = f_b * cb + i_b * g_b
                hb = o_b * jnp.tanh(cb)
                ysf_ref[:, pl.ds(tf, 1), :] = hf.reshape(bg, 1, H)
                ysb_ref[:, pl.ds(tb, 1), :] = hb.reshape(bg, 1, H)
                return jnp.concatenate([hf, hb], axis=1), cf, cb

            z = jnp.zeros((bg, H), f32)
            z2 = jnp.zeros((bg, 2 * H), f32)
            lax.fori_loop(0, L_MHC, step, (z2, z, z))

        x2d = x_ref[...].reshape(bg * L_MHC, MHC_DIM)
        run_layer(x2d, *ws[0])
        l1_ref[...] = jnp.concatenate([ysf_ref[...], ysb_ref[...]], axis=-1)
        l1_2d = l1_ref[...].reshape(bg * L_MHC, 2 * H)
        run_layer(l1_2d, *ws[1])
        o_ref[...] = jnp.concatenate([ysf_ref[...], ysb_ref[...]], axis=-1)

    in_specs = [pl.BlockSpec((bg, L_MHC, MHC_DIM), lambda i: (i, 0, 0))]
    for tup in weights:
        for w in tup:
            in_specs += [pl.BlockSpec(w.shape,
                                      (lambda i: (0, 0)) if w.ndim == 2
                                      else (lambda i: (0,)))]

    return pl.pallas_call(
        body,
        grid=(B // bg,),
        in_specs=in_specs,
        out_specs=pl.BlockSpec((bg, L_MHC, MHC_DIM), lambda i: (i, 0, 0)),
        out_shape=jax.ShapeDtypeStruct((B, L_MHC, MHC_DIM), f32),
        scratch_shapes=[
            pltpu.VMEM((bg, L_MHC, 4 * H), f32),
            pltpu.VMEM((bg, L_MHC, 4 * H), f32),
            pltpu.VMEM((bg, L_MHC, H), f32),
            pltpu.VMEM((bg, L_MHC, H), f32),
            pltpu.VMEM((bg, L_MHC, 2 * H), f32),
        ],
    )(mhc, *w_args)


def _tc_head(mhc_out, pep, fc1_t, fc1_b, out_wt, out_b, g_blk=8):
    def body(m_ref, p_ref, f1_ref, fb_ref, ow_ref, ob_ref, clus_ref, out_ref):
        a = m_ref[...].reshape(g_blk * L_MHC, MHC_DIM)
        bmat = p_ref[...].reshape(g_blk * N_PEP, MHC_DIM)
        full = lax.dot_general(a, bmat, (((1,), (1,)), ((), ())),
                               preferred_element_type=f32)
        parts = [full[L_MHC * g:L_MHC * (g + 1), N_PEP * g:N_PEP * (g + 1)]
                 for g in range(g_blk)]
        clus = jnp.stack(parts, axis=0)
        clus_ref[...] = clus
        xcin = clus.reshape(g_blk, L_MHC * N_PEP)
        xc = jax.nn.relu(jnp.dot(xcin, f1_ref[...], preferred_element_type=f32)
                         + fb_ref[...][None, :])
        out_ref[...] = (jnp.dot(xc, ow_ref[...], preferred_element_type=f32)
                        + ob_ref[...][None, :])

    return pl.pallas_call(
        body,
        grid=(B // g_blk,),
        in_specs=[pl.BlockSpec((g_blk, L_MHC, MHC_DIM), lambda i: (i, 0, 0)),
                  pl.BlockSpec((g_blk, N_PEP, MHC_DIM), lambda i: (i, 0, 0)),
                  pl.BlockSpec((L_MHC * N_PEP, 16), lambda i: (0, 0)),
                  pl.BlockSpec((16,), lambda i: (0,)),
                  pl.BlockSpec((16, 1), lambda i: (0, 0)),
                  pl.BlockSpec((1,), lambda i: (0,))],
        out_specs=[pl.BlockSpec((g_blk, L_MHC, N_PEP), lambda i: (i, 0, 0)),
                   pl.BlockSpec((g_blk, 1), lambda i: (i, 0))],
        out_shape=[jax.ShapeDtypeStruct((B, L_MHC, N_PEP), f32),
                   jax.ShapeDtypeStruct((B, 1), f32)],
    )(mhc_out, pep, fc1_t, fc1_b, out_wt, out_b)


def _fold(a):
    return a.reshape(NSLAB, N4, 4 * SLAB)


def _unfold(a):
    return a.reshape(NSLAB, N_NODES, SLAB)


def kernel(x, edge_index, mhc_embed, batch, params):
    node_idx = edge_index[0]
    he_idx = edge_index[1]
    nidx2d = node_idx.reshape(IDX_ROWS, IDXW)
    hidx2d = he_idx.reshape(IDX_ROWS, IDXW)

    ones128 = jnp.ones((IDXW,), f32)
    zeros3072 = jnp.zeros((STRIPE,), f32)
    zeros_stripe = jnp.zeros((STRIPE, SLAB), f32)

    dinv, binv = _sc_counts(nidx2d, hidx2d, ones128, zeros3072)
    binv_f = binv.reshape(N4, 4)

    b1_slab = params['hc1_b'].reshape(NSLAB, SLAB)
    b2_slab = params['hc2_b'].reshape(NSLAB, SLAB)

    xt1 = _tc_mm_slab(x, params['hc1_W'].T)
    m1 = _sc_segsum(xt1, nidx2d, hidx2d, zeros_stripe)
    m1s = _tc_scale(_fold(m1), binv_f)
    g1 = _sc_segsum(_unfold(m1s), hidx2d, nidx2d, zeros_stripe)
    xt2 = _tc_actmm_slab(g1, dinv, b1_slab, params['hc2_W'].T)
    m2 = _sc_segsum(xt2, nidx2d, hidx2d, zeros_stripe)
    m2s = _tc_scale(_fold(m2), binv_f)
    g2 = _sc_segsum(_unfold(m2s), hidx2d, nidx2d, zeros_stripe)
    pep_flat = _tc_pepmm(g2, dinv, b2_slab, params['pep_fc_W'].T,
                         params['pep_fc_b'])
    pep = pep_flat.reshape(B, N_PEP, MHC_DIM)

    weights = []
    for l in (0, 1):
        whh_fb = jnp.zeros((2 * H, 8 * H), f32)
        whh_fb = whh_fb.at[0:H, 0:4 * H].set(params['l%d_Whh_f' % l].T)
        whh_fb = whh_fb.at[H:2 * H, 4 * H:8 * H].set(params['l%d_Whh_b' % l].T)
        weights.append((
            params['l%d_Wih_f' % l].T,
            params['l%d_Wih_b' % l].T,
            whh_fb,
            params['l%d_bih_f' % l] + params['l%d_bhh_f' % l],
            params['l%d_bih_b' % l] + params['l%d_bhh_b' % l],
        ))
    mhc_out = _tc_bilstm(mhc_embed, weights)

    clus, out = _tc_head(mhc_out, pep, params['fc1_W'].T, params['fc1_b'],
                         params['out_W'].T, params['out_b'])
    return out, clus


# speedup vs baseline: 2.2731x; 1.2411x over previous
import functools

import jax
import jax.numpy as jnp
from jax import lax
from jax.experimental import pallas as pl
from jax.experimental.pallas import tpu as pltpu
from jax.experimental.pallas import tpu_sc as plsc

B = 2048
L_MHC = 34
N_PEP = 24
MHC_DIM = 128
H = 64
PEP_DIM = 320
N_NODES = B * N_PEP
N4 = N_NODES // 4
N_INC = 196608
NSLAB = 10
SLAB = 32
NC, NS = 2, 16
IDXW = 128
IDX_ROWS = N_INC // IDXW
ROWS_PER_TILE = IDX_ROWS // NS
STRIPE = N_NODES // NS

f32 = jnp.float32
i32 = jnp.int32


def _sc_counts(nidx2d, hidx2d, ones128, zeros3072):
    mesh = plsc.VectorSubcoreMesh(
        core_axis_name="c", subcore_axis_name="s", num_cores=NC, num_subcores=NS)

    @functools.partial(
        pl.kernel,
        out_type=(jax.ShapeDtypeStruct((N_NODES,), f32),
                  jax.ShapeDtypeStruct((N_NODES,), f32)),
        mesh=mesh,
        scratch_types=[
            pltpu.VMEM((ROWS_PER_TILE, IDXW), i32),
            pltpu.VMEM((IDXW,), f32),
            pltpu.VMEM((STRIPE,), f32),
            pltpu.VMEM_SHARED((N_NODES,), f32),
        ],
        compiler_params=pltpu.CompilerParams(use_tc_tiling_on_sc=False),
    )
    def counts_kernel(nidx_hbm, hidx_hbm, ones_hbm, zeros_hbm,
                      dinv_hbm, binv_hbm, idx_v, ones_v, buf_v, acc_sh):
        c = lax.axis_index("c")
        s = lax.axis_index("s")

        pltpu.sync_copy(ones_hbm, ones_v)

        def one_dir(idx_hbm, out_hbm):
            pltpu.sync_copy(zeros_hbm, buf_v)
            pltpu.sync_copy(buf_v, acc_sh.at[pl.ds(s * STRIPE, STRIPE)])
            pltpu.sync_copy(idx_hbm.at[pl.ds(s * ROWS_PER_TILE, ROWS_PER_TILE)],
                            idx_v)
            plsc.subcore_barrier()

            def body(j, carry):
                pltpu.sync_copy(ones_v, acc_sh.at[idx_v.at[j]], add=True)
                return carry
            lax.fori_loop(0, ROWS_PER_TILE, body, 0)
            plsc.subcore_barrier()

            pltpu.sync_copy(acc_sh.at[pl.ds(s * STRIPE, STRIPE)], buf_v)

            def recip(i, carry):
                v = buf_v[pl.ds(i * 16, 16)]
                buf_v[pl.ds(i * 16, 16)] = jnp.where(v > 0.0, 1.0 / v, 0.0)
                return carry
            lax.fori_loop(0, STRIPE // 16, recip, 0)
            pltpu.sync_copy(buf_v, out_hbm.at[pl.ds(s * STRIPE, STRIPE)])

        @pl.when(c == 0)
        def _():
            one_dir(nidx_hbm, dinv_hbm)

        @pl.when(c == 1)
        def _():
            one_dir(hidx_hbm, binv_hbm)

    return counts_kernel(nidx2d, hidx2d, ones128, zeros3072)


def _sc_segsum(src_slab, gidx2d, sidx2d, zeros_stripe):
    mesh = plsc.VectorSubcoreMesh(
        core_axis_name="c", subcore_axis_name="s", num_cores=NC, num_subcores=NS)

    @functools.partial(
        pl.kernel,
        out_type=jax.ShapeDtypeStruct((NSLAB, N_NODES, SLAB), f32),
        mesh=mesh,
        scratch_types=[
            pltpu.VMEM((ROWS_PER_TILE, IDXW), i32),
            pltpu.VMEM((ROWS_PER_TILE, IDXW), i32),
            pltpu.VMEM((IDXW, SLAB), f32),
            pltpu.VMEM((IDXW, SLAB), f32),
            pltpu.VMEM_SHARED((N_NODES, SLAB), f32),
            pltpu.SemaphoreType.DMA,
            pltpu.SemaphoreType.DMA,
        ],
        compiler_params=pltpu.CompilerParams(use_tc_tiling_on_sc=False),
    )
    def segsum_kernel(src_hbm, gidx_hbm, sidx_hbm, zst_hbm, out_hbm,
                      gidx_v, sidx_v, rows0_v, rows1_v, acc_sh, sem0, sem1):
        c = lax.axis_index("c")
        s = lax.axis_index("s")

        pltpu.sync_copy(gidx_hbm.at[pl.ds(s * ROWS_PER_TILE, ROWS_PER_TILE)],
                        gidx_v)
        pltpu.sync_copy(sidx_hbm.at[pl.ds(s * ROWS_PER_TILE, ROWS_PER_TILE)],
                        sidx_v)

        def run_slab(src_sl, out_sl):
            pltpu.sync_copy(zst_hbm, acc_sh.at[pl.ds(s * STRIPE, STRIPE)])
            plsc.subcore_barrier()

            rows = (rows0_v, rows1_v)
            sems = (sem0, sem1)

            def gather_start(j, b):
                pltpu.make_async_copy(
                    src_sl.at[gidx_v.at[j]], rows[b], sems[b]).start()

            def gather_wait(j, b):
                pltpu.make_async_copy(
                    src_sl.at[gidx_v.at[j]], rows[b], sems[b]).wait()

            gather_start(0, 0)
            gather_start(1, 1)

            def body(it, carry):
                for bpar in (0, 1):
                    j = it * 2 + bpar
                    gather_wait(j, bpar)

                    @pl.when(j + 2 < ROWS_PER_TILE)
                    def _():
                        gather_start(j + 2, bpar)
                    pltpu.sync_copy(rows[bpar], acc_sh.at[sidx_v.at[j]],
                                    add=True)
                return carry
            lax.fori_loop(0, ROWS_PER_TILE // 2, body, 0)
            plsc.subcore_barrier()
            pltpu.sync_copy(acc_sh.at[pl.ds(s * STRIPE, STRIPE)],
                            out_sl.at[pl.ds(s * STRIPE, STRIPE)])
            plsc.subcore_barrier()

        for half in range(NSLAB // NC):
            for cc in range(NC):
                slab = half * NC + cc

                @pl.when(c == cc)
                def _(slab=slab):
                    run_slab(src_hbm.at[slab], out_hbm.at[slab])

    return segsum_kernel(src_slab, gidx2d, sidx2d, zeros_stripe)


def _tc_mm_slab(x, wt, bm=512):
    def body(x_ref, w_ref, o_ref):
        acc = jnp.dot(x_ref[...], w_ref[...], preferred_element_type=f32)
        for sl in range(NSLAB):
            o_ref[sl] = acc[:, SLAB * sl:SLAB * (sl + 1)]

    return pl.pallas_call(
        body,
        grid=(N_NODES // bm,),
        in_specs=[pl.BlockSpec((bm, PEP_DIM), lambda i: (i, 0)),
                  pl.BlockSpec((PEP_DIM, PEP_DIM), lambda i: (0, 0))],
        out_specs=pl.BlockSpec((NSLAB, bm, SLAB), lambda i: (0, i, 0)),
        out_shape=jax.ShapeDtypeStruct((NSLAB, N_NODES, SLAB), f32),
    )(x, wt)


def _tc_scale(m, scale_f, br4=1024):
    def body(m_ref, s_ref, o_ref):
        sf = s_ref[...]
        srow = jnp.concatenate(
            [jnp.broadcast_to(sf[:, q:q + 1], (br4, SLAB)) for q in range(4)],
            axis=1)
        for sl in range(NSLAB):
            o_ref[sl] = m_ref[sl] * srow

    return pl.pallas_call(
        body,
        grid=(N4 // br4,),
        in_specs=[pl.BlockSpec((NSLAB, br4, 4 * SLAB), lambda i: (0, i, 0)),
                  pl.BlockSpec((br4, 4), lambda i: (i, 0))],
        out_specs=pl.BlockSpec((NSLAB, br4, 4 * SLAB), lambda i: (0, i, 0)),
        out_shape=jax.ShapeDtypeStruct((NSLAB, N4, 4 * SLAB), f32),
    )(m, scale_f)


def _srow_from_folded(sf, n_rows):
    return jnp.concatenate(
        [jnp.broadcast_to(sf[:, q:q + 1], (n_rows, SLAB)) for q in range(4)],
        axis=1)


def _tc_actmm_fold(g, dinv_f, b_row, w4, bm4=256):
    def body(g_ref, d_ref, b_ref, w_ref, o_ref):
        drow = _srow_from_folded(d_ref[...], bm4)
        parts = [jax.nn.relu(g_ref[sl] * drow + b_ref[sl][None, :])
                 for sl in range(NSLAB)]
        h4 = jnp.concatenate(parts, axis=1)
        acc = jnp.dot(h4, w_ref[...], preferred_element_type=f32)
        for sl in range(NSLAB):
            o_ref[sl] = acc[:, 4 * SLAB * sl:4 * SLAB * (sl + 1)]

    return pl.pallas_call(
        body,
        grid=(N4 // bm4,),
        in_specs=[pl.BlockSpec((NSLAB, bm4, 4 * SLAB), lambda i: (0, i, 0)),
                  pl.BlockSpec((bm4, 4), lambda i: (i, 0)),
                  pl.BlockSpec((NSLAB, 4 * SLAB), lambda i: (0, 0)),
                  pl.BlockSpec((4 * PEP_DIM, 4 * PEP_DIM), lambda i: (0, 0))],
        out_specs=pl.BlockSpec((NSLAB, bm4, 4 * SLAB), lambda i: (0, i, 0)),
        out_shape=jax.ShapeDtypeStruct((NSLAB, N4, 4 * SLAB), f32),
    )(g, dinv_f, b_row, w4)


def _tc_pepmm_fold(g, dinv_f, b_row, w4p, pb4, bm4=256):
    def body(g_ref, d_ref, b_ref, w_ref, pb_ref, o_ref):
        drow = _srow_from_folded(d_ref[...], bm4)
        parts = [g_ref[sl] * drow + b_ref[sl][None, :] for sl in range(NSLAB)]
        h4 = jnp.concatenate(parts, axis=1)
        o_ref[...] = (jnp.dot(h4, w_ref[...], preferred_element_type=f32)
                      + pb_ref[...][None, :])

    return pl.pallas_call(
        body,
        grid=(N4 // bm4,),
        in_specs=[pl.BlockSpec((NSLAB, bm4, 4 * SLAB), lambda i: (0, i, 0)),
                  pl.BlockSpec((bm4, 4), lambda i: (i, 0)),
                  pl.BlockSpec((NSLAB, 4 * SLAB), lambda i: (0, 0)),
                  pl.BlockSpec((4 * PEP_DIM, 4 * MHC_DIM), lambda i: (0, 0)),
                  pl.BlockSpec((4 * MHC_DIM,), lambda i: (0,))],
        out_specs=pl.BlockSpec((bm4, 4 * MHC_DIM), lambda i: (i, 0)),
        out_shape=jax.ShapeDtypeStruct((N4, 4 * MHC_DIM), f32),
    )(g, dinv_f, b_row, w4p, pb4)


def _sigmoid(x):
    return 1.0 / (1.0 + jnp.exp(-x))


def _tc_bilstm(mhc, weights, bg=256):
    w_args = []
    for tup in weights:
        w_args += list(tup)

    def body(x_ref, wf0, wb0, wh0, bf0, bb0, wf1, wb1, wh1, bf1, bb1,
             o_ref, gxf_ref, gxb_ref, ysf_ref, ysb_ref, l1_ref):
        ws = [(wf0, wb0, wh0, bf0, bb0), (wf1, wb1, wh1, bf1, bb1)]

        def run_layer(in2d, wihf_t, wihb_t, whh_fb, bsf, bsb):
            gxf = jnp.dot(in2d, wihf_t[...], preferred_element_type=f32)
            gxf_ref[...] = (gxf + bsf[...][None, :]).reshape(bg, L_MHC, 4 * H)
            gxb = jnp.dot(in2d, wihb_t[...], preferred_element_type=f32)
            gxb_ref[...] = (gxb + bsb[...][None, :]).reshape(bg, L_MHC, 4 * H)

            def step(k, carry):
                h2, cf, cb = carry
                tf = k
                tb = L_MHC - 1 - k
                gf = gxf_ref[:, pl.ds(tf, 1), :].reshape(bg, 4 * H)
                gb = gxb_ref[:, pl.ds(tb, 1), :].reshape(bg, 4 * H)
                g = jnp.concatenate([gf, gb], axis=1)
                g = g + jnp.dot(h2, whh_fb[...], preferred_element_type=f32)
                i_f = _sigmoid(g[:, 0:H])
                f_f = _sigmoid(g[:, H:2 * H])
                g_f = jnp.tanh(g[:, 2 * H:3 * H])
                o_f = _sigmoid(g[:, 3 * H:4 * H])
                i_b = _sigmoid(g[:, 4 * H:5 * H])
                f_b = _sigmoid(g[:, 5 * H:6 * H])
                g_b = jnp.tanh(g[:, 6 * H:7 * H])
                o_b = _sigmoid(g[:, 7 * H:8 * H])
                cf = f_f * cf + i_f * g_f
                hf = o_f * jnp.tanh(cf)
                cb = f_b * cb + i_b * g_b
                hb = o_b * jnp.tanh(cb)
                ysf_ref[:, pl.ds(tf, 1), :] = hf.reshape(bg, 1, H)
                ysb_ref[:, pl.ds(tb, 1), :] = hb.reshape(bg, 1, H)
                return jnp.concatenate([hf, hb], axis=1), cf, cb

            z = jnp.zeros((bg, H), f32)
            z2 = jnp.zeros((bg, 2 * H), f32)
            lax.fori_loop(0, L_MHC, step, (z2, z, z))

        x2d = x_ref[...].reshape(bg * L_MHC, MHC_DIM)
        run_layer(x2d, *ws[0])
        l1_ref[...] = jnp.concatenate([ysf_ref[...], ysb_ref[...]], axis=-1)
        l1_2d = l1_ref[...].reshape(bg * L_MHC, 2 * H)
        run_layer(l1_2d, *ws[1])
        o_ref[...] = jnp.concatenate([ysf_ref[...], ysb_ref[...]], axis=-1)

    in_specs = [pl.BlockSpec((bg, L_MHC, MHC_DIM), lambda i: (i, 0, 0))]
    for tup in weights:
        for w in tup:
            in_specs += [pl.BlockSpec(w.shape,
                                      (lambda i: (0, 0)) if w.ndim == 2
                                      else (lambda i: (0,)))]

    return pl.pallas_call(
        body,
        grid=(B // bg,),
        in_specs=in_specs,
        out_specs=pl.BlockSpec((bg, L_MHC, MHC_DIM), lambda i: (i, 0, 0)),
        out_shape=jax.ShapeDtypeStruct((B, L_MHC, MHC_DIM), f32),
        scratch_shapes=[
            pltpu.VMEM((bg, L_MHC, 4 * H), f32),
            pltpu.VMEM((bg, L_MHC, 4 * H), f32),
            pltpu.VMEM((bg, L_MHC, H), f32),
            pltpu.VMEM((bg, L_MHC, H), f32),
            pltpu.VMEM((bg, L_MHC, 2 * H), f32),
        ],
    )(mhc, *w_args)


def _tc_head(mhc_out, pep, fc1_t, fc1_b, out_wt, out_b, g_blk=8):
    def body(m_ref, p_ref, f1_ref, fb_ref, ow_ref, ob_ref, clus_ref, out_ref):
        a = m_ref[...].reshape(g_blk * L_MHC, MHC_DIM)
        bmat = p_ref[...].reshape(g_blk * N_PEP, MHC_DIM)
        full = lax.dot_general(a, bmat, (((1,), (1,)), ((), ())),
                               preferred_element_type=f32)
        parts = [full[L_MHC * g:L_MHC * (g + 1), N_PEP * g:N_PEP * (g + 1)]
                 for g in range(g_blk)]
        clus = jnp.stack(parts, axis=0)
        clus_ref[...] = clus
        xcin = clus.reshape(g_blk, L_MHC * N_PEP)
        xc = jax.nn.relu(jnp.dot(xcin, f1_ref[...], preferred_element_type=f32)
                         + fb_ref[...][None, :])
        out_ref[...] = (jnp.dot(xc, ow_ref[...], preferred_element_type=f32)
                        + ob_ref[...][None, :])

    return pl.pallas_call(
        body,
        grid=(B // g_blk,),
        in_specs=[pl.BlockSpec((g_blk, L_MHC, MHC_DIM), lambda i: (i, 0, 0)),
                  pl.BlockSpec((g_blk, N_PEP, MHC_DIM), lambda i: (i, 0, 0)),
                  pl.BlockSpec((L_MHC * N_PEP, 16), lambda i: (0, 0)),
                  pl.BlockSpec((16,), lambda i: (0,)),
                  pl.BlockSpec((16, 1), lambda i: (0, 0)),
                  pl.BlockSpec((1,), lambda i: (0,))],
        out_specs=[pl.BlockSpec((g_blk, L_MHC, N_PEP), lambda i: (i, 0, 0)),
                   pl.BlockSpec((g_blk, 1), lambda i: (i, 0))],
        out_shape=[jax.ShapeDtypeStruct((B, L_MHC, N_PEP), f32),
                   jax.ShapeDtypeStruct((B, 1), f32)],
    )(mhc_out, pep, fc1_t, fc1_b, out_wt, out_b)


def _fold(a):
    return a.reshape(NSLAB, N4, 4 * SLAB)


def _unfold(a):
    return a.reshape(NSLAB, N_NODES, SLAB)


def kernel(x, edge_index, mhc_embed, batch, params):
    node_idx = edge_index[0]
    he_idx = edge_index[1]
    nidx2d = node_idx.reshape(IDX_ROWS, IDXW)
    hidx2d = he_idx.reshape(IDX_ROWS, IDXW)

    ones128 = jnp.ones((IDXW,), f32)
    zeros3072 = jnp.zeros((STRIPE,), f32)
    zeros_stripe = jnp.zeros((STRIPE, SLAB), f32)

    dinv, binv = _sc_counts(nidx2d, hidx2d, ones128, zeros3072)
    binv_f = binv.reshape(N4, 4)
    dinv_f = dinv.reshape(N4, 4)

    b1_row = jnp.tile(params['hc1_b'].reshape(NSLAB, 1, SLAB),
                      (1, 4, 1)).reshape(NSLAB, 4 * SLAB)
    b2_row = jnp.tile(params['hc2_b'].reshape(NSLAB, 1, SLAB),
                      (1, 4, 1)).reshape(NSLAB, 4 * SLAB)

    eye4 = jnp.eye(4, dtype=f32)
    wt2 = params['hc2_W'].T.reshape(NSLAB, SLAB, NSLAB, SLAB)
    w4 = (wt2[:, None, :, :, None, :]
          * eye4[None, :, None, None, :, None]).reshape(4 * PEP_DIM,
                                                        4 * PEP_DIM)
    pwt = params['pep_fc_W'].T.reshape(NSLAB, SLAB, MHC_DIM)
    w4p = (pwt[:, None, :, None, :]
           * eye4[None, :, None, :, None]).reshape(4 * PEP_DIM, 4 * MHC_DIM)
    pb4 = jnp.tile(params['pep_fc_b'], 4)

    xt1 = _tc_mm_slab(x, params['hc1_W'].T)
    m1 = _sc_segsum(xt1, nidx2d, hidx2d, zeros_stripe)
    m1s = _tc_scale(_fold(m1), binv_f)
    g1 = _sc_segsum(_unfold(m1s), hidx2d, nidx2d, zeros_stripe)
    xt2 = _tc_actmm_fold(_fold(g1), dinv_f, b1_row, w4)
    m2 = _sc_segsum(_unfold(xt2), nidx2d, hidx2d, zeros_stripe)
    m2s = _tc_scale(_fold(m2), binv_f)
    g2 = _sc_segsum(_unfold(m2s), hidx2d, nidx2d, zeros_stripe)
    pep4 = _tc_pepmm_fold(_fold(g2), dinv_f, b2_row, w4p, pb4)
    pep = pep4.reshape(B, N_PEP, MHC_DIM)

    weights = []
    for l in (0, 1):
        whh_fb = jnp.zeros((2 * H, 8 * H), f32)
        whh_fb = whh_fb.at[0:H, 0:4 * H].set(params['l%d_Whh_f' % l].T)
        whh_fb = whh_fb.at[H:2 * H, 4 * H:8 * H].set(params['l%d_Whh_b' % l].T)
        weights.append((
            params['l%d_Wih_f' % l].T,
            params['l%d_Wih_b' % l].T,
            whh_fb,
            params['l%d_bih_f' % l] + params['l%d_bhh_f' % l],
            params['l%d_bih_b' % l] + params['l%d_bhh_b' % l],
        ))
    mhc_out = _tc_bilstm(mhc_embed, weights)

    clus, out = _tc_head(mhc_out, pep, params['fc1_W'].T, params['fc1_b'],
                         params['out_W'].T, params['out_b'])
    return out, clus
